# baseline (speedup 1.0000x reference)
import functools

import jax
import jax.numpy as jnp
from jax import lax
from jax.experimental import pallas as pl
from jax.experimental.pallas import tpu as pltpu
from jax.experimental.pallas import tpu_sc as plsc

D = 32
S = 50
L = 16
BLK = 512
CHUNK = 128


def _info():
    info = plsc.get_sparse_core_info()
    return info.num_cores, info.num_subcores


def _bcast(v):
    return lax.full((L,), v, jnp.int32)


@functools.lru_cache(maxsize=None)
def _make_repack(V):
    NC, NS = _info()
    NW = NC * NS
    nblk = V // BLK
    rem = V - nblk * BLK
    slots = (nblk + NW - 1) // NW

    mesh = plsc.VectorSubcoreMesh(core_axis_name="c", subcore_axis_name="s")

    @functools.partial(
        pl.kernel,
        mesh=mesh,
        out_type=jax.ShapeDtypeStruct((V * D // 128, 128), jnp.float32),
        scratch_types=[
            pltpu.VMEM((D, BLK), jnp.float32),
            pltpu.VMEM((BLK // 4, 128), jnp.float32),
        ],
        compiler_params=pltpu.CompilerParams(needs_layout_passes=False),
    )
    def k(tT_hbm, tail_hbm, r_hbm, in_v, out_v):
        wid = lax.axis_index("s") * NC + lax.axis_index("c")
        iota = lax.iota(jnp.int32, L)

        def transpose_block(g, carry):
            for m in range(8):
                q = m // 2
                rows = iota + (L * (m % 2))
                cols = _bcast(4 * g + q)
                vals = plsc.load_gather(in_v, [rows, cols])
                plsc.store_scatter(out_v, [_bcast(g), iota + L * m], vals)
            return carry

        def run(t, carry):
            bi = wid + NW * t

            @pl.when(bi < nblk)
            def _():
                c0 = pl.multiple_of(bi * BLK, BLK)
                pltpu.sync_copy(tT_hbm.at[:, pl.ds(c0, BLK)], in_v)
                lax.fori_loop(0, BLK // 4, transpose_block, 0)
                g0 = pl.multiple_of(bi * (BLK // 4), BLK // 4)
                pltpu.sync_copy(out_v, r_hbm.at[pl.ds(g0, BLK // 4)])
            return carry

        lax.fori_loop(0, slots, run, 0)

        if rem:
            @pl.when(wid == NW - 1)
            def _():
                nt = rem // 4
                pltpu.sync_copy(tail_hbm, out_v.at[pl.ds(0, nt)])
                pltpu.sync_copy(out_v.at[pl.ds(0, nt)],
                                r_hbm.at[pl.ds(V * D // 128 - nt, nt)])

    return k


@functools.lru_cache(maxsize=None)
def _make_gather(B, V):
    NC, NS = _info()
    NW = NC * NS
    bpw = B // NW
    npw = bpw * S
    nch = npw // CHUNK
    jmax = bpw // CHUNK

    mesh = plsc.VectorSubcoreMesh(core_axis_name="c", subcore_axis_name="s")

    @functools.partial(
        pl.kernel,
        mesh=mesh,
        out_type=jax.ShapeDtypeStruct((S, D, B), jnp.float32),
        scratch_types=[
            pltpu.VMEM((npw,), jnp.int32),
            pltpu.VMEM((CHUNK,), jnp.int32),
            pltpu.VMEM((CHUNK, CHUNK), jnp.float32),
            pltpu.VMEM((D, CHUNK), jnp.float32),
            pltpu.SemaphoreType.DMA,
        ],
        compiler_params=pltpu.CompilerParams(needs_layout_passes=False),
    )
    def k(idx_hbm, r_hbm, out_hbm, xv, gv, big, tbuf, gsem):
        wid = lax.axis_index("s") * NC + lax.axis_index("c")
        iota = lax.iota(jnp.int32, L)
        i0 = pl.multiple_of(wid * npw, npw)
        pltpu.sync_copy(idx_hbm.at[pl.ds(i0, npw)], xv)

        def chunk(c, carry):
            s = c // jmax
            j = c % jmax
            for kk in range(CHUNK // L):
                flat = (iota + (L * kk + CHUNK * j)) * S + s
                v = plsc.load_gather(xv, [flat])
                plsc.store_scatter(gv, [iota + L * kk],
                                   lax.shift_right_logical(v, 2))
            pltpu.async_copy(r_hbm.at[gv], big, gsem).wait()
            for kk in range(CHUNK // L):
                rows = iota + (L * kk)
                flat = (iota + (L * kk + CHUNK * j)) * S + s
                q = lax.bitwise_and(plsc.load_gather(xv, [flat]), 3)
                colb = q * D
                for f in range(D):
                    vals = plsc.load_gather(big, [rows, colb + f])
                    plsc.store_scatter(tbuf, [_bcast(f), rows], vals)
            bcol = pl.multiple_of(wid * bpw + j * CHUNK, CHUNK)
            pltpu.sync_copy(tbuf, out_hbm.at[s, :, pl.ds(bcol, CHUNK)])
            return carry

        lax.fori_loop(0, nch, chunk, 0)

    return k


def kernel(x, table):
    B = x.shape[0]
    V = table.shape[0]
    tT = table.T
    rem = V - (V // BLK) * BLK
    tail = table[V - rem:].reshape(rem // 4, 128)
    R = _make_repack(V)(tT, tail)
    idx = x.reshape(B * S)
    out3 = _make_gather(B, V)(idx, R)
    return jnp.transpose(out3, (2, 0, 1))


# speedup vs baseline: 1.3664x; 1.3664x over previous
import functools

import jax
import jax.numpy as jnp
from jax import lax
from jax.experimental import pallas as pl
from jax.experimental.pallas import tpu as pltpu
from jax.experimental.pallas import tpu_sc as plsc

D = 32
S = 50
L = 16
BLK = 512
CHUNK = 128


def _info():
    info = plsc.get_sparse_core_info()
    return info.num_cores, info.num_subcores


def _bcast(v):
    return lax.full((L,), v, jnp.int32)


@functools.lru_cache(maxsize=None)
def _make_repack(V):
    NC, NS = _info()
    NW = NC * NS
    nblk = V // BLK
    rem = V - nblk * BLK
    slots = (nblk + NW - 1) // NW

    mesh = plsc.VectorSubcoreMesh(core_axis_name="c", subcore_axis_name="s")

    @functools.partial(
        pl.kernel,
        mesh=mesh,
        out_type=jax.ShapeDtypeStruct((V * D // 128, 128), jnp.float32),
        scratch_types=[
            pltpu.VMEM((D, BLK), jnp.float32),
            pltpu.VMEM((D, BLK), jnp.float32),
            pltpu.VMEM((BLK // 4, 128), jnp.float32),
            pltpu.VMEM((BLK // 4, 128), jnp.float32),
            pltpu.SemaphoreType.DMA,
            pltpu.SemaphoreType.DMA,
            pltpu.SemaphoreType.DMA,
            pltpu.SemaphoreType.DMA,
        ],
        compiler_params=pltpu.CompilerParams(needs_layout_passes=False),
    )
    def k(tT_hbm, tail_hbm, r_hbm, in0, in1, out0, out1, is0, is1, os0, os1):
        wid = lax.axis_index("s") * NC + lax.axis_index("c")
        iota = lax.iota(jnp.int32, L)
        fvecs = [_bcast(f) for f in range(D)]

        ins = (in0, in1)
        outs = (out0, out1)
        isems = (is0, is1)
        osems = (os0, os1)

        def fire_in(t, p):
            bi = wid + NW * t
            c0 = pl.multiple_of(bi * BLK, BLK)
            pltpu.async_copy(tT_hbm.at[:, pl.ds(c0, BLK)], ins[p], isems[p])

        def transpose_block(p):
            def cgroup(cg, carry):
                cvec = iota + cg * L
                rows = lax.shift_right_logical(cvec, 2)
                colb = lax.bitwise_and(cvec, 3) * D
                for f in range(D):
                    vals = plsc.load_gather(ins[p], [fvecs[f], cvec])
                    plsc.store_scatter(outs[p], [rows, colb + f], vals)
                return carry
            lax.fori_loop(0, BLK // L, cgroup, 0)

        def drain_in(p):
            pltpu.make_async_copy(
                tT_hbm.at[:, pl.ds(0, BLK)], ins[p], isems[p]).wait()

        def fire_out(t, p):
            bi = wid + NW * t
            g0 = pl.multiple_of(bi * (BLK // 4), BLK // 4)
            pltpu.async_copy(outs[p], r_hbm.at[pl.ds(g0, BLK // 4)], osems[p])

        def drain_out(p):
            pltpu.make_async_copy(
                outs[p], r_hbm.at[pl.ds(0, BLK // 4)], osems[p]).wait()

        fire_in(0, 0)

        def run(u, carry):
            for p in (0, 1):
                t = 2 * u + p
                bi = wid + NW * t

                @pl.when(bi < nblk)
                def _(t=t, p=p):
                    @pl.when(wid + NW * (t + 1) < nblk)
                    def _():
                        fire_in(t + 1, 1 - p)
                    drain_in(p)
                    transpose_block(p)

                    @pl.when(t >= 2)
                    def _():
                        drain_out(p)
                    fire_out(t, p)
            return carry

        lax.fori_loop(0, (slots + 1) // 2, run, 0)

        drain_out(0)
        drain_out(1)

        if rem:
            @pl.when(wid == NW - 1)
            def _():
                nt = rem // 4
                pltpu.sync_copy(tail_hbm, out0.at[pl.ds(0, nt)])
                pltpu.sync_copy(out0.at[pl.ds(0, nt)],
                                r_hbm.at[pl.ds(V * D // 128 - nt, nt)])

    return k


@functools.lru_cache(maxsize=None)
def _make_gather(B, V):
    NC, NS = _info()
    NW = NC * NS
    bpw = B // NW
    npw = bpw * S
    nch = npw // CHUNK
    jmax = bpw // CHUNK

    mesh = plsc.VectorSubcoreMesh(core_axis_name="c", subcore_axis_name="s")

    @functools.partial(
        pl.kernel,
        mesh=mesh,
        out_type=jax.ShapeDtypeStruct((S, D, B), jnp.float32),
        scratch_types=[
            pltpu.VMEM((npw,), jnp.int32),
            pltpu.VMEM((CHUNK,), jnp.int32),
            pltpu.VMEM((CHUNK,), jnp.int32),
            pltpu.VMEM((CHUNK, CHUNK), jnp.float32),
            pltpu.VMEM((CHUNK, CHUNK), jnp.float32),
            pltpu.VMEM((D, CHUNK), jnp.float32),
            pltpu.VMEM((D, CHUNK), jnp.float32),
            pltpu.SemaphoreType.DMA,
            pltpu.SemaphoreType.DMA,
            pltpu.SemaphoreType.DMA,
            pltpu.SemaphoreType.DMA,
        ],
        compiler_params=pltpu.CompilerParams(needs_layout_passes=False),
    )
    def k(idx_hbm, r_hbm, out_hbm, xv, gv0, gv1, big0, big1, tb0, tb1,
          gs0, gs1, ts0, ts1):
        wid = lax.axis_index("s") * NC + lax.axis_index("c")
        iota = lax.iota(jnp.int32, L)
        rowvecs = [iota + L * kk for kk in range(CHUNK // L)]
        fvecs = [_bcast(f) for f in range(D)]
        gvs = (gv0, gv1)
        bigs = (big0, big1)
        tbs = (tb0, tb1)
        gsems = (gs0, gs1)
        tsems = (ts0, ts1)

        i0 = pl.multiple_of(wid * npw, npw)
        pltpu.sync_copy(idx_hbm.at[pl.ds(i0, npw)], xv)

        def fire_gather(c, p):
            s = c // jmax
            j = c % jmax
            for kk in range(CHUNK // L):
                flat = (rowvecs[kk] + CHUNK * j) * S + s
                v = plsc.load_gather(xv, [flat])
                plsc.store_scatter(gvs[p], [rowvecs[kk]],
                                   lax.shift_right_logical(v, 2))
            pltpu.async_copy(r_hbm.at[gvs[p]], bigs[p], gsems[p])

        def drain_gather(p):
            pltpu.make_async_copy(
                r_hbm.at[pl.ds(0, CHUNK)], bigs[p], gsems[p]).wait()

        def process(c, p):
            s = c // jmax
            j = c % jmax
            for kk in range(CHUNK // L):
                flat = (rowvecs[kk] + CHUNK * j) * S + s
                q = lax.bitwise_and(plsc.load_gather(xv, [flat]), 3)
                colb = q * D
                for f in range(D):
                    vals = plsc.load_gather(bigs[p], [rowvecs[kk], colb + f])
                    plsc.store_scatter(tbs[p], [fvecs[f], rowvecs[kk]], vals)
            bcol = pl.multiple_of(wid * bpw + j * CHUNK, CHUNK)
            pltpu.async_copy(tbs[p], out_hbm.at[s, :, pl.ds(bcol, CHUNK)],
                             tsems[p])

        def drain_store(p):
            pltpu.make_async_copy(
                tbs[p], out_hbm.at[0, :, pl.ds(0, CHUNK)], tsems[p]).wait()

        fire_gather(0, 0)

        def run(h, carry):
            for p in (0, 1):
                c = 2 * h + p

                @pl.when(c + 1 < nch)
                def _(c=c, p=p):
                    fire_gather(c + 1, 1 - p)
                drain_gather(p)

                @pl.when(c >= 2)
                def _(p=p):
                    drain_store(p)
                process(c, p)
            return carry

        lax.fori_loop(0, nch // 2, run, 0)
        drain_store(0)
        drain_store(1)

    return k


def kernel(x, table):
    B = x.shape[0]
    V = table.shape[0]
    tT = table.T
    rem = V - (V // BLK) * BLK
    tail = table[V - rem:].reshape(rem // 4, 128)
    R = _make_repack(V)(tT, tail)
    idx = x.reshape(B * S)
    out3 = _make_gather(B, V)(idx, R)
    return jnp.transpose(out3, (2, 0, 1))


# speedup vs baseline: 3.3771x; 2.4715x over previous
import functools

import jax
import jax.numpy as jnp
from jax import lax
from jax.experimental import pallas as pl
from jax.experimental.pallas import tpu as pltpu
from jax.experimental.pallas import tpu_sc as plsc

D = 32
S = 50
L = 16
BLK = 512
CHUNK = 128


def _info():
    info = plsc.get_sparse_core_info()
    return info.num_cores, info.num_subcores


def _bcast(v):
    return lax.full((L,), v, jnp.int32)


@functools.lru_cache(maxsize=None)
def _make_repack(V):
    NC, NS = _info()
    NW = NC * NS
    nblk = V // BLK
    rem = V - nblk * BLK
    slots = (nblk + NW - 1) // NW

    mesh = plsc.VectorSubcoreMesh(core_axis_name="c", subcore_axis_name="s")

    @functools.partial(
        pl.kernel,
        mesh=mesh,
        out_type=jax.ShapeDtypeStruct((V * D // 128, 128), jnp.float32),
        scratch_types=[
            pltpu.VMEM((D, BLK), jnp.float32),
            pltpu.VMEM((D, BLK), jnp.float32),
            pltpu.VMEM((BLK // 4, 128), jnp.float32),
            pltpu.VMEM((BLK // 4, 128), jnp.float32),
            pltpu.SemaphoreType.DMA,
            pltpu.SemaphoreType.DMA,
            pltpu.SemaphoreType.DMA,
            pltpu.SemaphoreType.DMA,
        ],
        compiler_params=pltpu.CompilerParams(needs_layout_passes=False),
    )
    def k(tT_hbm, tail_hbm, r_hbm, in0, in1, out0, out1, is0, is1, os0, os1):
        wid = lax.axis_index("s") * NC + lax.axis_index("c")
        iota = lax.iota(jnp.int32, L)
        dvecs = [lax.rem(iota + k, L) for k in range(L)]

        ins = (in0, in1)
        outs = (out0, out1)
        isems = (is0, is1)
        osems = (os0, os1)

        def fire_in(t, p):
            bi = wid + NW * t
            c0 = pl.multiple_of(bi * BLK, BLK)
            pltpu.async_copy(tT_hbm.at[:, pl.ds(c0, BLK)], ins[p], isems[p])

        def transpose_block(p):
            def cgroup(cg, carry):
                cvec = iota + cg * L
                rows = lax.shift_right_logical(cvec, 2)
                qc = lax.bitwise_and(cvec, 3) * D
                for e in range(D // L):
                    for k in range(L):
                        fv = dvecs[k] + L * e
                        vals = plsc.load_gather(ins[p], [fv, cvec])
                        plsc.store_scatter(outs[p], [rows, qc + fv], vals)
                return carry
            lax.fori_loop(0, BLK // L, cgroup, 0)

        def drain_in(p):
            pltpu.make_async_copy(
                tT_hbm.at[:, pl.ds(0, BLK)], ins[p], isems[p]).wait()

        def fire_out(t, p):
            bi = wid + NW * t
            g0 = pl.multiple_of(bi * (BLK // 4), BLK // 4)
            pltpu.async_copy(outs[p], r_hbm.at[pl.ds(g0, BLK // 4)], osems[p])

        def drain_out(p):
            pltpu.make_async_copy(
                outs[p], r_hbm.at[pl.ds(0, BLK // 4)], osems[p]).wait()

        fire_in(0, 0)

        def run(u, carry):
            for p in (0, 1):
                t = 2 * u + p
                bi = wid + NW * t

                @pl.when(bi < nblk)
                def _(t=t, p=p):
                    @pl.when(wid + NW * (t + 1) < nblk)
                    def _():
                        fire_in(t + 1, 1 - p)
                    drain_in(p)
                    transpose_block(p)

                    @pl.when(t >= 2)
                    def _():
                        drain_out(p)
                    fire_out(t, p)
            return carry

        lax.fori_loop(0, (slots + 1) // 2, run, 0)

        drain_out(0)
        drain_out(1)

        if rem:
            @pl.when(wid == NW - 1)
            def _():
                nt = rem // 4
                pltpu.sync_copy(tail_hbm, out0.at[pl.ds(0, nt)])
                pltpu.sync_copy(out0.at[pl.ds(0, nt)],
                                r_hbm.at[pl.ds(V * D // 128 - nt, nt)])

    return k


@functools.lru_cache(maxsize=None)
def _make_gather(B, V):
    NC, NS = _info()
    NW = NC * NS
    bpw = B // NW
    npw = bpw * S
    nch = npw // CHUNK
    jmax = bpw // CHUNK

    mesh = plsc.VectorSubcoreMesh(core_axis_name="c", subcore_axis_name="s")

    @functools.partial(
        pl.kernel,
        mesh=mesh,
        out_type=jax.ShapeDtypeStruct((S, D, B), jnp.float32),
        scratch_types=[
            pltpu.VMEM((npw,), jnp.int32),
            pltpu.VMEM((CHUNK,), jnp.int32),
            pltpu.VMEM((CHUNK,), jnp.int32),
            pltpu.VMEM((CHUNK, CHUNK), jnp.float32),
            pltpu.VMEM((CHUNK, CHUNK), jnp.float32),
            pltpu.VMEM((D, CHUNK), jnp.float32),
            pltpu.VMEM((D, CHUNK), jnp.float32),
            pltpu.SemaphoreType.DMA,
            pltpu.SemaphoreType.DMA,
            pltpu.SemaphoreType.DMA,
            pltpu.SemaphoreType.DMA,
        ],
        compiler_params=pltpu.CompilerParams(needs_layout_passes=False),
    )
    def k(idx_hbm, r_hbm, out_hbm, xv, gv0, gv1, big0, big1, tb0, tb1,
          gs0, gs1, ts0, ts1):
        wid = lax.axis_index("s") * NC + lax.axis_index("c")
        iota = lax.iota(jnp.int32, L)
        rowvecs = [iota + L * kk for kk in range(CHUNK // L)]
        dvecs = [lax.rem(iota + k, L) for k in range(L)]
        gvs = (gv0, gv1)
        bigs = (big0, big1)
        tbs = (tb0, tb1)
        gsems = (gs0, gs1)
        tsems = (ts0, ts1)

        i0 = pl.multiple_of(wid * npw, npw)
        pltpu.sync_copy(idx_hbm.at[pl.ds(i0, npw)], xv)

        def fire_gather(c, p):
            s = c // jmax
            j = c % jmax
            for kk in range(CHUNK // L):
                flat = (rowvecs[kk] + CHUNK * j) * S + s
                v = plsc.load_gather(xv, [flat])
                plsc.store_scatter(gvs[p], [rowvecs[kk]],
                                   lax.shift_right_logical(v, 2))
            pltpu.async_copy(r_hbm.at[gvs[p]], bigs[p], gsems[p])

        def drain_gather(p):
            pltpu.make_async_copy(
                r_hbm.at[pl.ds(0, CHUNK)], bigs[p], gsems[p]).wait()

        def process(c, p):
            s = c // jmax
            j = c % jmax

            def kgroup(kk, carry):
                rowvec = iota + L * kk
                flat = (rowvec + CHUNK * j) * S + s
                q = lax.bitwise_and(plsc.load_gather(xv, [flat]), 3)
                colb = q * D
                for e in range(D // L):
                    for k in range(L):
                        fv = dvecs[k] + L * e
                        vals = plsc.load_gather(bigs[p], [rowvec, colb + fv])
                        plsc.store_scatter(tbs[p], [fv, rowvec], vals)
                return carry

            lax.fori_loop(0, CHUNK // L, kgroup, 0)
            bcol = pl.multiple_of(wid * bpw + j * CHUNK, CHUNK)
            pltpu.async_copy(tbs[p], out_hbm.at[s, :, pl.ds(bcol, CHUNK)],
                             tsems[p])

        def drain_store(p):
            pltpu.make_async_copy(
                tbs[p], out_hbm.at[0, :, pl.ds(0, CHUNK)], tsems[p]).wait()

        fire_gather(0, 0)

        def run(h, carry):
            for p in (0, 1):
                c = 2 * h + p

                @pl.when(c + 1 < nch)
                def _(c=c, p=p):
                    fire_gather(c + 1, 1 - p)
                drain_gather(p)

                @pl.when(c >= 2)
                def _(p=p):
                    drain_store(p)
                process(c, p)
            return carry

        lax.fori_loop(0, nch // 2, run, 0)
        drain_store(0)
        drain_store(1)

    return k


def kernel(x, table):
    B = x.shape[0]
    V = table.shape[0]
    tT = table.T
    rem = V - (V // BLK) * BLK
    tail = table[V - rem:].reshape(rem // 4, 128)
    R = _make_repack(V)(tT, tail)
    idx = x.reshape(B * S)
    out3 = _make_gather(B, V)(idx, R)
    return jnp.transpose(out3, (2, 0, 1))


# speedup vs baseline: 3.4129x; 1.0106x over previous
import functools

import jax
import jax.numpy as jnp
from jax import lax
from jax.experimental import pallas as pl
from jax.experimental.pallas import tpu as pltpu
from jax.experimental.pallas import tpu_sc as plsc

D = 32
S = 50
L = 16
BLK = 512
CHUNK = 128


def _info():
    info = plsc.get_sparse_core_info()
    return info.num_cores, info.num_subcores


def _bcast(v):
    return lax.full((L,), v, jnp.int32)


@functools.lru_cache(maxsize=None)
def _make_repack(V):
    NC, NS = _info()
    NW = NC * NS
    nblk = V // BLK
    rem = V - nblk * BLK
    slots = (nblk + NW - 1) // NW

    mesh = plsc.VectorSubcoreMesh(core_axis_name="c", subcore_axis_name="s")

    @functools.partial(
        pl.kernel,
        mesh=mesh,
        out_type=jax.ShapeDtypeStruct((V * D // 128, 128), jnp.float32),
        scratch_types=[
            pltpu.VMEM((D, BLK), jnp.float32),
            pltpu.VMEM((D, BLK), jnp.float32),
            pltpu.VMEM((BLK // 4, 128), jnp.float32),
            pltpu.VMEM((BLK // 4, 128), jnp.float32),
            pltpu.SemaphoreType.DMA,
            pltpu.SemaphoreType.DMA,
            pltpu.SemaphoreType.DMA,
            pltpu.SemaphoreType.DMA,
        ],
        compiler_params=pltpu.CompilerParams(needs_layout_passes=False),
    )
    def k(tT_hbm, tail_hbm, r_hbm, in0, in1, out0, out1, is0, is1, os0, os1):
        wid = lax.axis_index("s") * NC + lax.axis_index("c")
        iota = lax.iota(jnp.int32, L)
        dvecs = [lax.rem(iota + k, L) for k in range(L)]

        ins = (in0, in1)
        outs = (out0, out1)
        isems = (is0, is1)
        osems = (os0, os1)

        def fire_in(t, p):
            bi = wid + NW * t
            c0 = pl.multiple_of(bi * BLK, BLK)
            pltpu.async_copy(tT_hbm.at[:, pl.ds(c0, BLK)], ins[p], isems[p])

        def transpose_block(p):
            def cgroup(cg, carry):
                for d in range(2):
                    cvec = iota + (2 * cg + d) * L
                    rows = lax.shift_right_logical(cvec, 2)
                    qc = lax.bitwise_and(cvec, 3) * D
                    for e in range(D // L):
                        for k in range(L):
                            fv = dvecs[k] + L * e
                            vals = plsc.load_gather(ins[p], [fv, cvec])
                            plsc.store_scatter(outs[p], [rows, qc + fv], vals)
                return carry
            lax.fori_loop(0, BLK // L // 2, cgroup, 0)

        def drain_in(p):
            pltpu.make_async_copy(
                tT_hbm.at[:, pl.ds(0, BLK)], ins[p], isems[p]).wait()

        def fire_out(t, p):
            bi = wid + NW * t
            g0 = pl.multiple_of(bi * (BLK // 4), BLK // 4)
            pltpu.async_copy(outs[p], r_hbm.at[pl.ds(g0, BLK // 4)], osems[p])

        def drain_out(p):
            pltpu.make_async_copy(
                outs[p], r_hbm.at[pl.ds(0, BLK // 4)], osems[p]).wait()

        fire_in(0, 0)

        def run(u, carry):
            for p in (0, 1):
                t = 2 * u + p
                bi = wid + NW * t

                @pl.when(bi < nblk)
                def _(t=t, p=p):
                    @pl.when(wid + NW * (t + 1) < nblk)
                    def _():
                        fire_in(t + 1, 1 - p)
                    drain_in(p)
                    transpose_block(p)

                    @pl.when(t >= 2)
                    def _():
                        drain_out(p)
                    fire_out(t, p)
            return carry

        lax.fori_loop(0, (slots + 1) // 2, run, 0)

        drain_out(0)
        drain_out(1)

        if rem:
            @pl.when(wid == NW - 1)
            def _():
                nt = rem // 4
                pltpu.sync_copy(tail_hbm, out0.at[pl.ds(0, nt)])
                pltpu.sync_copy(out0.at[pl.ds(0, nt)],
                                r_hbm.at[pl.ds(V * D // 128 - nt, nt)])

    return k


@functools.lru_cache(maxsize=None)
def _make_gather(B, V):
    NC, NS = _info()
    NW = NC * NS
    bpw = B // NW
    npw = bpw * S
    nch = npw // CHUNK
    jmax = bpw // CHUNK

    mesh = plsc.VectorSubcoreMesh(core_axis_name="c", subcore_axis_name="s")

    @functools.partial(
        pl.kernel,
        mesh=mesh,
        out_type=jax.ShapeDtypeStruct((S, D, B), jnp.float32),
        scratch_types=[
            pltpu.VMEM((npw,), jnp.int32),
            pltpu.VMEM((CHUNK,), jnp.int32),
            pltpu.VMEM((CHUNK,), jnp.int32),
            pltpu.VMEM((CHUNK, CHUNK), jnp.float32),
            pltpu.VMEM((CHUNK, CHUNK), jnp.float32),
            pltpu.VMEM((D, CHUNK), jnp.float32),
            pltpu.VMEM((D, CHUNK), jnp.float32),
            pltpu.SemaphoreType.DMA,
            pltpu.SemaphoreType.DMA,
            pltpu.SemaphoreType.DMA,
            pltpu.SemaphoreType.DMA,
        ],
        compiler_params=pltpu.CompilerParams(needs_layout_passes=False),
    )
    def k(idx_hbm, r_hbm, out_hbm, xv, gv0, gv1, big0, big1, tb0, tb1,
          gs0, gs1, ts0, ts1):
        wid = lax.axis_index("s") * NC + lax.axis_index("c")
        iota = lax.iota(jnp.int32, L)
        rowvecs = [iota + L * kk for kk in range(CHUNK // L)]
        dvecs = [lax.rem(iota + k, L) for k in range(L)]
        gvs = (gv0, gv1)
        bigs = (big0, big1)
        tbs = (tb0, tb1)
        gsems = (gs0, gs1)
        tsems = (ts0, ts1)

        i0 = pl.multiple_of(wid * npw, npw)
        pltpu.sync_copy(idx_hbm.at[pl.ds(i0, npw)], xv)

        def fire_gather(c, p):
            s = c // jmax
            j = c % jmax
            for kk in range(CHUNK // L):
                flat = (rowvecs[kk] + CHUNK * j) * S + s
                v = plsc.load_gather(xv, [flat])
                plsc.store_scatter(gvs[p], [rowvecs[kk]],
                                   lax.shift_right_logical(v, 2))
            pltpu.async_copy(r_hbm.at[gvs[p]], bigs[p], gsems[p])

        def drain_gather(p):
            pltpu.make_async_copy(
                r_hbm.at[pl.ds(0, CHUNK)], bigs[p], gsems[p]).wait()

        def process(c, p):
            s = c // jmax
            j = c % jmax

            def kgroup(kk, carry):
                for d in range(2):
                    rowvec = iota + L * (2 * kk + d)
                    flat = (rowvec + CHUNK * j) * S + s
                    q = lax.bitwise_and(plsc.load_gather(xv, [flat]), 3)
                    colb = q * D
                    for e in range(D // L):
                        for k in range(L):
                            fv = dvecs[k] + L * e
                            vals = plsc.load_gather(bigs[p],
                                                    [rowvec, colb + fv])
                            plsc.store_scatter(tbs[p], [fv, rowvec], vals)
                return carry

            lax.fori_loop(0, CHUNK // L // 2, kgroup, 0)
            bcol = pl.multiple_of(wid * bpw + j * CHUNK, CHUNK)
            pltpu.async_copy(tbs[p], out_hbm.at[s, :, pl.ds(bcol, CHUNK)],
                             tsems[p])

        def drain_store(p):
            pltpu.make_async_copy(
                tbs[p], out_hbm.at[0, :, pl.ds(0, CHUNK)], tsems[p]).wait()

        fire_gather(0, 0)

        def run(h, carry):
            for p in (0, 1):
                c = 2 * h + p

                @pl.when(c + 1 < nch)
                def _(c=c, p=p):
                    fire_gather(c + 1, 1 - p)
                drain_gather(p)

                @pl.when(c >= 2)
                def _(p=p):
                    drain_store(p)
                process(c, p)
            return carry

        lax.fori_loop(0, nch // 2, run, 0)
        drain_store(0)
        drain_store(1)

    return k


def kernel(x, table):
    B = x.shape[0]
    V = table.shape[0]
    tT = table.T
    rem = V - (V // BLK) * BLK
    tail = table[V - rem:].reshape(rem // 4, 128)
    R = _make_repack(V)(tT, tail)
    idx = x.reshape(B * S)
    out3 = _make_gather(B, V)(idx, R)
    return jnp.transpose(out3, (2, 0, 1))


# speedup vs baseline: 5.1960x; 1.5225x over previous
import functools

import jax
import jax.numpy as jnp
from jax import lax
from jax.experimental import pallas as pl
from jax.experimental.pallas import tpu as pltpu
from jax.experimental.pallas import tpu_sc as plsc

D = 32
S = 50
L = 16
BLK = 512
CHUNK = 128


def _info():
    info = plsc.get_sparse_core_info()
    return info.num_cores, info.num_subcores


def _bcast(v):
    return lax.full((L,), v, jnp.int32)


@functools.lru_cache(maxsize=None)
def _make_repack(V):
    NC, NS = _info()
    NW = NC * NS
    nblk = V // BLK
    rem = V - nblk * BLK
    slots = (nblk + NW - 1) // NW

    mesh = plsc.VectorSubcoreMesh(core_axis_name="c", subcore_axis_name="s")

    @functools.partial(
        pl.kernel,
        mesh=mesh,
        out_type=jax.ShapeDtypeStruct((V * D // 128, 128), jnp.float32),
        scratch_types=[
            pltpu.VMEM((D, BLK), jnp.float32),
            pltpu.VMEM((D, BLK), jnp.float32),
            pltpu.VMEM((BLK // 4, 128), jnp.float32),
            pltpu.VMEM((BLK // 4, 128), jnp.float32),
            pltpu.SemaphoreType.DMA,
            pltpu.SemaphoreType.DMA,
            pltpu.SemaphoreType.DMA,
            pltpu.SemaphoreType.DMA,
        ],
        compiler_params=pltpu.CompilerParams(needs_layout_passes=False),
    )
    def k(tT_hbm, tail_hbm, r_hbm, in0, in1, out0, out1, is0, is1, os0, os1):
        wid = lax.axis_index("s") * NC + lax.axis_index("c")
        iota = lax.iota(jnp.int32, L)
        dvecs = [lax.rem(iota + k, L) for k in range(L)]

        ins = (in0, in1)
        outs = (out0, out1)
        isems = (is0, is1)
        osems = (os0, os1)

        def fire_in(t, p):
            bi = wid + NW * t
            c0 = pl.multiple_of(bi * BLK, BLK)
            pltpu.async_copy(tT_hbm.at[:, pl.ds(c0, BLK)], ins[p], isems[p])

        def transpose_block(p):
            def cgroup(cg, carry):
                for d in range(2):
                    cvec = iota + (2 * cg + d) * L
                    rows = lax.shift_right_logical(cvec, 2)
                    qc = lax.bitwise_and(cvec, 3) * D
                    for e in range(D // L):
                        for k8 in range(L // 8):
                            fvs = [dvecs[8 * k8 + u] + L * e for u in range(8)]
                            vals = [plsc.load_gather(ins[p], [fv, cvec])
                                    for fv in fvs]
                            for fv, v in zip(fvs, vals):
                                plsc.store_scatter(outs[p], [rows, qc + fv], v)
                return carry
            lax.fori_loop(0, BLK // L // 2, cgroup, 0)

        def drain_in(p):
            pltpu.make_async_copy(
                tT_hbm.at[:, pl.ds(0, BLK)], ins[p], isems[p]).wait()

        def fire_out(t, p):
            bi = wid + NW * t
            g0 = pl.multiple_of(bi * (BLK // 4), BLK // 4)
            pltpu.async_copy(outs[p], r_hbm.at[pl.ds(g0, BLK // 4)], osems[p])

        def drain_out(p):
            pltpu.make_async_copy(
                outs[p], r_hbm.at[pl.ds(0, BLK // 4)], osems[p]).wait()

        fire_in(0, 0)

        def run(u, carry):
            for p in (0, 1):
                t = 2 * u + p
                bi = wid + NW * t

                @pl.when(bi < nblk)
                def _(t=t, p=p):
                    @pl.when(wid + NW * (t + 1) < nblk)
                    def _():
                        fire_in(t + 1, 1 - p)
                    drain_in(p)
                    transpose_block(p)

                    @pl.when(t >= 2)
                    def _():
                        drain_out(p)
                    fire_out(t, p)
            return carry

        lax.fori_loop(0, (slots + 1) // 2, run, 0)

        drain_out(0)
        drain_out(1)

        if rem:
            @pl.when(wid == NW - 1)
            def _():
                nt = rem // 4
                pltpu.sync_copy(tail_hbm, out0.at[pl.ds(0, nt)])
                pltpu.sync_copy(out0.at[pl.ds(0, nt)],
                                r_hbm.at[pl.ds(V * D // 128 - nt, nt)])

    return k


@functools.lru_cache(maxsize=None)
def _make_gather(B, V):
    NC, NS = _info()
    NW = NC * NS
    bpw = B // NW
    npw = bpw * S
    nch = npw // CHUNK
    jmax = bpw // CHUNK

    mesh = plsc.VectorSubcoreMesh(core_axis_name="c", subcore_axis_name="s")

    @functools.partial(
        pl.kernel,
        mesh=mesh,
        out_type=jax.ShapeDtypeStruct((S, D, B), jnp.float32),
        scratch_types=[
            pltpu.VMEM((npw,), jnp.int32),
            pltpu.VMEM((CHUNK,), jnp.int32),
            pltpu.VMEM((CHUNK,), jnp.int32),
            pltpu.VMEM((CHUNK, CHUNK), jnp.float32),
            pltpu.VMEM((CHUNK, CHUNK), jnp.float32),
            pltpu.VMEM((D, CHUNK), jnp.float32),
            pltpu.VMEM((D, CHUNK), jnp.float32),
            pltpu.SemaphoreType.DMA,
            pltpu.SemaphoreType.DMA,
            pltpu.SemaphoreType.DMA,
            pltpu.SemaphoreType.DMA,
        ],
        compiler_params=pltpu.CompilerParams(needs_layout_passes=False),
    )
    def k(idx_hbm, r_hbm, out_hbm, xv, gv0, gv1, big0, big1, tb0, tb1,
          gs0, gs1, ts0, ts1):
        wid = lax.axis_index("s") * NC + lax.axis_index("c")
        iota = lax.iota(jnp.int32, L)
        rowvecs = [iota + L * kk for kk in range(CHUNK // L)]
        dvecs = [lax.rem(iota + k, L) for k in range(L)]
        gvs = (gv0, gv1)
        bigs = (big0, big1)
        tbs = (tb0, tb1)
        gsems = (gs0, gs1)
        tsems = (ts0, ts1)

        i0 = pl.multiple_of(wid * npw, npw)
        pltpu.sync_copy(idx_hbm.at[pl.ds(i0, npw)], xv)

        def fire_gather(c, p):
            s = c // jmax
            j = c % jmax
            for kk in range(CHUNK // L):
                flat = (rowvecs[kk] + CHUNK * j) * S + s
                v = plsc.load_gather(xv, [flat])
                plsc.store_scatter(gvs[p], [rowvecs[kk]],
                                   lax.shift_right_logical(v, 2))
            pltpu.async_copy(r_hbm.at[gvs[p]], bigs[p], gsems[p])

        def drain_gather(p):
            pltpu.make_async_copy(
                r_hbm.at[pl.ds(0, CHUNK)], bigs[p], gsems[p]).wait()

        def process(c, p):
            s = c // jmax
            j = c % jmax

            def kgroup(kk, carry):
                for d in range(2):
                    rowvec = iota + L * (2 * kk + d)
                    flat = (rowvec + CHUNK * j) * S + s
                    q = lax.bitwise_and(plsc.load_gather(xv, [flat]), 3)
                    colb = q * D
                    for e in range(D // L):
                        for k8 in range(L // 8):
                            fvs = [dvecs[8 * k8 + u] + L * e for u in range(8)]
                            vals = [plsc.load_gather(bigs[p],
                                                     [rowvec, colb + fv])
                                    for fv in fvs]
                            for fv, v in zip(fvs, vals):
                                plsc.store_scatter(tbs[p], [fv, rowvec], v)
                return carry

            lax.fori_loop(0, CHUNK // L // 2, kgroup, 0)
            bcol = pl.multiple_of(wid * bpw + j * CHUNK, CHUNK)
            pltpu.async_copy(tbs[p], out_hbm.at[s, :, pl.ds(bcol, CHUNK)],
                             tsems[p])

        def drain_store(p):
            pltpu.make_async_copy(
                tbs[p], out_hbm.at[0, :, pl.ds(0, CHUNK)], tsems[p]).wait()

        fire_gather(0, 0)

        def run(h, carry):
            for p in (0, 1):
                c = 2 * h + p

                @pl.when(c + 1 < nch)
                def _(c=c, p=p):
                    fire_gather(c + 1, 1 - p)
                drain_gather(p)

                @pl.when(c >= 2)
                def _(p=p):
                    drain_store(p)
                process(c, p)
            return carry

        lax.fori_loop(0, nch // 2, run, 0)
        drain_store(0)
        drain_store(1)

    return k


def kernel(x, table):
    B = x.shape[0]
    V = table.shape[0]
    tT = table.T
    rem = V - (V // BLK) * BLK
    tail = table[V - rem:].reshape(rem // 4, 128)
    R = _make_repack(V)(tT, tail)
    idx = x.reshape(B * S)
    out3 = _make_gather(B, V)(idx, R)
    return jnp.transpose(out3, (2, 0, 1))
